# baseline (speedup 1.0000x reference)
import functools

import jax
import jax.numpy as jnp
from jax import lax
from jax.experimental import pallas as pl
from jax.experimental.pallas import tpu as pltpu
from jax.experimental.pallas import tpu_sc as plsc

N = 10000
D = 128
DH = D // 2
E = 320000
NC = 2
NS = 16
NW = NC * NS
NPAD = 10240
RPT = NPAD // NS

KD = 128
EPWD = 10240
NCHD = EPWD // KD
DGRP = 5

K = 128
EPT = 20480
NCH = EPT // K
EGRP = 4

_mesh = plsc.VectorSubcoreMesh(
    core_axis_name="c", subcore_axis_name="s", num_cores=NC, num_subcores=NS
)


@functools.partial(
    pl.kernel,
    out_type=jax.ShapeDtypeStruct((NC * NPAD, 16), jnp.float32),
    mesh=_mesh,
    scratch_types=[
        [pltpu.VMEM((KD,), jnp.int32) for _ in range(2 * DGRP)],
        pltpu.VMEM((KD, 16), jnp.float32),
        pltpu.VMEM((128, 16), jnp.float32),
        pltpu.VMEM_SHARED((NPAD, 16), jnp.float32),
        pltpu.SemaphoreType.DMA,
    ],
    compiler_params=pltpu.CompilerParams(use_tc_tiling_on_sc=False),
)
def _deg_kernel(dst_hbm, degpart_hbm, dbufs, ones_v, zbuf, deg_sh, isem):
    c = lax.axis_index("c")
    s = lax.axis_index("s")
    wid = c * NS + s

    lane = lax.iota(jnp.int32, 16)
    onevec = jnp.where(lane == 0, 1.0, 0.0).astype(jnp.float32)
    zvec = jnp.zeros((16,), jnp.float32)

    def init_rows(r, _):
        ones_v[r, :] = onevec
        return 0

    lax.fori_loop(0, KD, init_rows, 0)

    def zrow(r, _):
        zbuf[r, :] = zvec
        return 0

    lax.fori_loop(0, 128, zrow, 0)

    for z in range(RPT // 128):
        pltpu.sync_copy(zbuf, deg_sh.at[pl.ds(s * RPT + z * 128, 128)])
    plsc.subcore_barrier()

    def ifire(t, half):
        for b in range(DGRP):
            pltpu.async_copy(
                dst_hbm.at[wid * NCHD + t * DGRP + b],
                dbufs[half * DGRP + b], isem,
            )

    def idrain(t, half):
        for b in range(DGRP):
            pltpu.make_async_copy(
                dst_hbm.at[wid * NCHD + t * DGRP + b],
                dbufs[half * DGRP + b], isem,
            ).wait()

    def scats(half):
        for b in range(DGRP):
            pltpu.sync_copy(
                ones_v, deg_sh.at[dbufs[half * DGRP + b]], add=True
            )

    ifire(0, 0)

    def outer(p, _):
        ifire(2 * p + 1, 1)
        idrain(2 * p, 0)
        scats(0)

        @pl.when(p < NCHD // (2 * DGRP) - 1)
        def _():
            ifire(2 * p + 2, 0)

        idrain(2 * p + 1, 1)
        scats(1)
        return 0

    lax.fori_loop(0, NCHD // (2 * DGRP), outer, 0)
    plsc.subcore_barrier()

    pltpu.sync_copy(
        deg_sh.at[pl.ds(s * RPT, RPT)],
        degpart_hbm.at[pl.ds(c * NPAD + s * RPT, RPT)],
    )


@functools.partial(
    pl.kernel,
    out_type=jax.ShapeDtypeStruct((NC * NPAD, DH), jnp.float32),
    mesh=_mesh,
    scratch_types=[
        pltpu.VMEM((NCH, K), jnp.int32),
        [pltpu.VMEM((K,), jnp.int32) for _ in range(2 * EGRP)],
        pltpu.VMEM((2, EGRP, K, DH), jnp.float32),
        pltpu.VMEM_SHARED((NPAD, DH), jnp.float32),
        pltpu.SemaphoreType.DMA,
        pltpu.SemaphoreType.DMA,
    ],
    compiler_params=pltpu.CompilerParams(use_tc_tiling_on_sc=False),
)
def _edge_kernel(y_hbm, src_hbm, dst_hbm, acc_hbm, src_v, dbufs,
                 rows_v, acc_sh, gsem, isem):
    c = lax.axis_index("c")
    s = lax.axis_index("s")
    wid = c * NS + s

    pltpu.sync_copy(src_hbm.at[s], src_v)

    offs = c * N

    def adj(r, _):
        for q in range(K // 16):
            sl = pl.ds(q * 16, 16)
            src_v[r, sl] = src_v[r, sl] + offs
        return 0

    lax.fori_loop(0, NCH, adj, 0)

    zvec = jnp.zeros((16,), jnp.float32)

    def zrow(r, _):
        for cc in range(DH // 16):
            rows_v[0, 0, r, pl.ds(cc * 16, 16)] = zvec
        return 0

    lax.fori_loop(0, K, zrow, 0)

    for z in range(RPT // K):
        pltpu.sync_copy(
            rows_v.at[0, 0], acc_sh.at[pl.ds(s * RPT + z * K, K)]
        )
    plsc.subcore_barrier()

    def gfire(t, half):
        for b in range(EGRP):
            j = t * EGRP + b
            pltpu.async_copy(
                dst_hbm.at[s * NCH + j], dbufs[half * EGRP + b], isem
            )
            pltpu.async_copy(
                y_hbm.at[src_v.at[j]], rows_v.at[half, b], gsem
            )

    def gdrain(t, half):
        for b in range(EGRP):
            j = t * EGRP + b
            pltpu.make_async_copy(
                dst_hbm.at[s * NCH + j], dbufs[half * EGRP + b], isem
            ).wait()
            pltpu.make_async_copy(
                y_hbm.at[src_v.at[j]], rows_v.at[half, b], gsem
            ).wait()

    def scats(half):
        for b in range(EGRP):
            pltpu.sync_copy(
                rows_v.at[half, b], acc_sh.at[dbufs[half * EGRP + b]],
                add=True,
            )

    gfire(0, 0)

    def outer(p, _):
        gfire(2 * p + 1, 1)
        gdrain(2 * p, 0)
        scats(0)

        @pl.when(p < NCH // (2 * EGRP) - 1)
        def _():
            gfire(2 * p + 2, 0)

        gdrain(2 * p + 1, 1)
        scats(1)
        return 0

    lax.fori_loop(0, NCH // (2 * EGRP), outer, 0)
    plsc.subcore_barrier()

    pltpu.sync_copy(
        acc_sh.at[pl.ds(s * RPT, RPT)],
        acc_hbm.at[pl.ds(c * NPAD + s * RPT, RPT)],
    )


BLK = 2000
GRID = N // BLK


def _mm_body(hn_ref, w_ref, x_ref):
    x_ref[...] = jnp.dot(
        hn_ref[...], w_ref[...], preferred_element_type=jnp.float32
    )


def _tc_matmul(hn, W):
    return pl.pallas_call(
        _mm_body,
        grid=(GRID,),
        in_specs=[
            pl.BlockSpec((BLK, D), lambda i: (i, 0)),
            pl.BlockSpec((D, D), lambda i: (0, 0)),
        ],
        out_specs=pl.BlockSpec((BLK, D), lambda i: (i, 0)),
        out_shape=jax.ShapeDtypeStruct((N, D), jnp.float32),
    )(hn, W)


def _scale_body(x_ref, dp_ref, y_ref):
    dp = dp_ref[...]
    deg = dp[0, :, 0:1] + dp[1, :, 0:1] + 1.0
    dinv = 1.0 / jnp.sqrt(deg)
    y = x_ref[...] * dinv
    y_ref[0, :, :] = y[:, :DH]
    y_ref[1, :, :] = y[:, DH:]


def _tc_scale(x, degpart):
    return pl.pallas_call(
        _scale_body,
        grid=(GRID,),
        in_specs=[
            pl.BlockSpec((BLK, D), lambda i: (i, 0)),
            pl.BlockSpec((NC, BLK, 16), lambda i: (0, i, 0)),
        ],
        out_specs=pl.BlockSpec((2, BLK, DH), lambda i: (0, i, 0)),
        out_shape=jax.ShapeDtypeStruct((2, N, DH), jnp.float32),
    )(x, degpart)


def _fin_body(p_ref, y_ref, dp_ref, b_ref, o_ref):
    dp = dp_ref[...]
    deg = dp[0, :, 0:1] + dp[1, :, 0:1] + 1.0
    dinv = 1.0 / jnp.sqrt(deg)
    acc = jnp.concatenate([p_ref[0], p_ref[1]], axis=1)
    y = jnp.concatenate([y_ref[0], y_ref[1]], axis=1)
    ssum = (acc + y) * dinv + b_ref[...]
    o_ref[...] = jnp.maximum(ssum, 0.0)


def _tc_finish(parts, y2, degpart, b2):
    return pl.pallas_call(
        _fin_body,
        grid=(GRID,),
        in_specs=[
            pl.BlockSpec((NC, BLK, DH), lambda i: (0, i, 0)),
            pl.BlockSpec((2, BLK, DH), lambda i: (0, i, 0)),
            pl.BlockSpec((NC, BLK, 16), lambda i: (0, i, 0)),
            pl.BlockSpec((1, D), lambda i: (0, 0)),
        ],
        out_specs=pl.BlockSpec((BLK, D), lambda i: (i, 0)),
        out_shape=jax.ShapeDtypeStruct((N, D), jnp.float32),
    )(parts, y2, degpart, b2)


def kernel(hn, edge_index, he, W, b):
    ei = edge_index.astype(jnp.int32)
    pad_d = jnp.full((NW, EPWD - E // NW), N, jnp.int32)
    dst3 = jnp.concatenate(
        [ei[1].reshape(NW, E // NW), pad_d], axis=1
    ).reshape(NW * NCHD, KD)
    pad_s = jnp.zeros((NS, EPT - E // NS), jnp.int32)
    pad_t = jnp.full((NS, EPT - E // NS), N, jnp.int32)
    src_e = jnp.concatenate(
        [ei[0].reshape(NS, E // NS), pad_s], axis=1
    ).reshape(NS, NCH, K)
    dst_e = jnp.concatenate(
        [ei[1].reshape(NS, E // NS), pad_t], axis=1
    ).reshape(NS * NCH, K)

    degpart = _deg_kernel(dst3).reshape(NC, NPAD, 16)
    y2 = _tc_scale(_tc_matmul(hn, W), degpart)
    parts = _edge_kernel(
        y2.reshape(2 * N, DH), src_e, dst_e
    ).reshape(NC, NPAD, DH)
    out = _tc_finish(parts, y2, degpart, b.reshape(1, D))
    return out


# speedup vs baseline: 1.6190x; 1.6190x over previous
import functools

import jax
import jax.numpy as jnp
from jax import lax
from jax.experimental import pallas as pl
from jax.experimental.pallas import tpu as pltpu
from jax.experimental.pallas import tpu_sc as plsc

N = 10000
D = 128
DH = D // 2
E = 320000
NC = 2
NS = 16
NW = NC * NS
NPAD = 10240
RPT = NPAD // NS

KD = 40
NCHD = E // NW // KD
DGRP = 5

K = 80
NCH = E // NS // K
EGRP = 5

_mesh = plsc.VectorSubcoreMesh(
    core_axis_name="c", subcore_axis_name="s", num_cores=NC, num_subcores=NS
)


@functools.partial(
    pl.kernel,
    out_type=jax.ShapeDtypeStruct((NC * NPAD, 16), jnp.float32),
    mesh=_mesh,
    scratch_types=[
        [pltpu.VMEM((KD,), jnp.int32) for _ in range(2 * DGRP)],
        pltpu.VMEM((KD, 16), jnp.float32),
        pltpu.VMEM((128, 16), jnp.float32),
        pltpu.VMEM_SHARED((NPAD, 16), jnp.float32),
        pltpu.SemaphoreType.DMA,
    ],
    compiler_params=pltpu.CompilerParams(use_tc_tiling_on_sc=False),
)
def _deg_kernel(dst_hbm, degpart_hbm, dbufs, ones_v, zbuf, deg_sh, isem):
    c = lax.axis_index("c")
    s = lax.axis_index("s")
    wid = c * NS + s

    lane = lax.iota(jnp.int32, 16)
    onevec = jnp.where(lane == 0, 1.0, 0.0).astype(jnp.float32)
    zvec = jnp.zeros((16,), jnp.float32)

    def init_rows(r, _):
        ones_v[r, :] = onevec
        return 0

    lax.fori_loop(0, KD, init_rows, 0)

    def zrow(r, _):
        zbuf[r, :] = zvec
        return 0

    lax.fori_loop(0, 128, zrow, 0)

    for z in range(RPT // 128):
        pltpu.sync_copy(zbuf, deg_sh.at[pl.ds(s * RPT + z * 128, 128)])
    plsc.subcore_barrier()

    def ifire(t, half):
        for b in range(DGRP):
            pltpu.async_copy(
                dst_hbm.at[wid * NCHD + t * DGRP + b],
                dbufs[half * DGRP + b], isem,
            )

    def idrain(t, half):
        for b in range(DGRP):
            pltpu.make_async_copy(
                dst_hbm.at[wid * NCHD + t * DGRP + b],
                dbufs[half * DGRP + b], isem,
            ).wait()

    def scats(half):
        for b in range(DGRP):
            pltpu.sync_copy(
                ones_v, deg_sh.at[dbufs[half * DGRP + b]], add=True
            )

    ifire(0, 0)

    def outer(p, _):
        ifire(2 * p + 1, 1)
        idrain(2 * p, 0)
        scats(0)

        @pl.when(p < NCHD // (2 * DGRP) - 1)
        def _():
            ifire(2 * p + 2, 0)

        idrain(2 * p + 1, 1)
        scats(1)
        return 0

    lax.fori_loop(0, NCHD // (2 * DGRP), outer, 0)
    plsc.subcore_barrier()

    pltpu.sync_copy(
        deg_sh.at[pl.ds(s * RPT, RPT)],
        degpart_hbm.at[pl.ds(c * NPAD + s * RPT, RPT)],
    )


@functools.partial(
    pl.kernel,
    out_type=jax.ShapeDtypeStruct((NC * NPAD, DH), jnp.float32),
    mesh=_mesh,
    scratch_types=[
        pltpu.VMEM((NCH, K), jnp.int32),
        [pltpu.VMEM((K,), jnp.int32) for _ in range(2 * EGRP)],
        pltpu.VMEM((2, EGRP, K, DH), jnp.float32),
        pltpu.VMEM_SHARED((NPAD, DH), jnp.float32),
        pltpu.SemaphoreType.DMA,
        pltpu.SemaphoreType.DMA,
    ],
    compiler_params=pltpu.CompilerParams(use_tc_tiling_on_sc=False),
)
def _edge_kernel(y_hbm, src_hbm, dst_hbm, acc_hbm, src_v, dbufs,
                 rows_v, acc_sh, gsem, isem):
    c = lax.axis_index("c")
    s = lax.axis_index("s")
    wid = c * NS + s

    pltpu.sync_copy(src_hbm.at[s], src_v)

    offs = c * N

    def adj(r, _):
        for q in range(K // 16):
            sl = pl.ds(q * 16, 16)
            src_v[r, sl] = src_v[r, sl] + offs
        return 0

    lax.fori_loop(0, NCH, adj, 0)

    zvec = jnp.zeros((16,), jnp.float32)

    def zrow(r, _):
        for cc in range(DH // 16):
            rows_v[0, 0, r, pl.ds(cc * 16, 16)] = zvec
        return 0

    lax.fori_loop(0, K, zrow, 0)

    for z in range(RPT // K):
        pltpu.sync_copy(
            rows_v.at[0, 0], acc_sh.at[pl.ds(s * RPT + z * K, K)]
        )
    plsc.subcore_barrier()

    def gfire(t, half):
        for b in range(EGRP):
            j = t * EGRP + b
            pltpu.async_copy(
                dst_hbm.at[s * NCH + j], dbufs[half * EGRP + b], isem
            )
            pltpu.async_copy(
                y_hbm.at[src_v.at[j]], rows_v.at[half, b], gsem
            )

    def gdrain(t, half):
        for b in range(EGRP):
            j = t * EGRP + b
            pltpu.make_async_copy(
                dst_hbm.at[s * NCH + j], dbufs[half * EGRP + b], isem
            ).wait()
            pltpu.make_async_copy(
                y_hbm.at[src_v.at[j]], rows_v.at[half, b], gsem
            ).wait()

    def scats(half):
        for b in range(EGRP):
            pltpu.sync_copy(
                rows_v.at[half, b], acc_sh.at[dbufs[half * EGRP + b]],
                add=True,
            )

    gfire(0, 0)

    def outer(p, _):
        gfire(2 * p + 1, 1)
        gdrain(2 * p, 0)
        scats(0)

        @pl.when(p < NCH // (2 * EGRP) - 1)
        def _():
            gfire(2 * p + 2, 0)

        gdrain(2 * p + 1, 1)
        scats(1)
        return 0

    lax.fori_loop(0, NCH // (2 * EGRP), outer, 0)
    plsc.subcore_barrier()

    pltpu.sync_copy(
        acc_sh.at[pl.ds(s * RPT, RPT)],
        acc_hbm.at[pl.ds(c * NPAD + s * RPT, RPT)],
    )


BLK = 2000
GRID = N // BLK


def _mm_body(hn_ref, w_ref, x_ref):
    x_ref[...] = jnp.dot(
        hn_ref[...], w_ref[...], preferred_element_type=jnp.float32
    )


def _tc_matmul(hn, W):
    return pl.pallas_call(
        _mm_body,
        grid=(GRID,),
        in_specs=[
            pl.BlockSpec((BLK, D), lambda i: (i, 0)),
            pl.BlockSpec((D, D), lambda i: (0, 0)),
        ],
        out_specs=pl.BlockSpec((BLK, D), lambda i: (i, 0)),
        out_shape=jax.ShapeDtypeStruct((N, D), jnp.float32),
    )(hn, W)


def _scale_body(x_ref, dp_ref, y_ref):
    dp = dp_ref[...]
    deg = dp[0, :, 0:1] + dp[1, :, 0:1] + 1.0
    dinv = 1.0 / jnp.sqrt(deg)
    y = x_ref[...] * dinv
    y_ref[0, :, :] = y[:, :DH]
    y_ref[1, :, :] = y[:, DH:]


def _tc_scale(x, degpart):
    return pl.pallas_call(
        _scale_body,
        grid=(GRID,),
        in_specs=[
            pl.BlockSpec((BLK, D), lambda i: (i, 0)),
            pl.BlockSpec((NC, BLK, 16), lambda i: (0, i, 0)),
        ],
        out_specs=pl.BlockSpec((2, BLK, DH), lambda i: (0, i, 0)),
        out_shape=jax.ShapeDtypeStruct((2, N, DH), jnp.float32),
    )(x, degpart)


def _fin_body(p_ref, y_ref, dp_ref, b_ref, o_ref):
    dp = dp_ref[...]
    deg = dp[0, :, 0:1] + dp[1, :, 0:1] + 1.0
    dinv = 1.0 / jnp.sqrt(deg)
    acc = jnp.concatenate([p_ref[0], p_ref[1]], axis=1)
    y = jnp.concatenate([y_ref[0], y_ref[1]], axis=1)
    ssum = (acc + y) * dinv + b_ref[...]
    o_ref[...] = jnp.maximum(ssum, 0.0)


def _tc_finish(parts, y2, degpart, b2):
    return pl.pallas_call(
        _fin_body,
        grid=(GRID,),
        in_specs=[
            pl.BlockSpec((NC, BLK, DH), lambda i: (0, i, 0)),
            pl.BlockSpec((2, BLK, DH), lambda i: (0, i, 0)),
            pl.BlockSpec((NC, BLK, 16), lambda i: (0, i, 0)),
            pl.BlockSpec((1, D), lambda i: (0, 0)),
        ],
        out_specs=pl.BlockSpec((BLK, D), lambda i: (i, 0)),
        out_shape=jax.ShapeDtypeStruct((N, D), jnp.float32),
    )(parts, y2, degpart, b2)


def kernel(hn, edge_index, he, W, b):
    ei = edge_index.astype(jnp.int32)
    dst3 = ei[1].reshape(NW * NCHD, KD)
    src_e = ei[0].reshape(NS, NCH, K)
    dst_e = ei[1].reshape(NS * NCH, K)

    degpart = _deg_kernel(dst3).reshape(NC, NPAD, 16)
    y2 = _tc_scale(_tc_matmul(hn, W), degpart)
    parts = _edge_kernel(
        y2.reshape(2 * N, DH), src_e, dst_e
    ).reshape(NC, NPAD, DH)
    out = _tc_finish(parts, y2, degpart, b.reshape(1, D))
    return out
